# baseline (speedup 1.0000x reference)
import functools

import jax
import jax.numpy as jnp
from jax import lax
from jax.experimental import pallas as pl
from jax.experimental.pallas import tpu as pltpu, tpu_sc as plsc

N = 10000
E = 320000
D = 128
G = 64

PAD_N = 10240
NC = 2
NS = 16
CB = 125
NCH = 80
IB = 16
NBLK = NCH // IB
IDXR = 2 * IB
EPW = NCH * CB
HALFROWS = NC * NS * NCH
RPT = PAD_N // NS

_sc_mesh = plsc.VectorSubcoreMesh(core_axis_name="c", subcore_axis_name="s")


@functools.partial(
    pl.kernel,
    out_type=jax.ShapeDtypeStruct((NC * PAD_N, D), jnp.float32),
    mesh=_sc_mesh,
    compiler_params=pltpu.CompilerParams(use_tc_tiling_on_sc=False),
    scratch_types=(
        pltpu.VMEM((IDXR, CB), jnp.int32),
        pltpu.VMEM((IDXR, CB), jnp.int32),
        pltpu.VMEM((CB, D), jnp.float32),
        pltpu.VMEM((CB, D), jnp.float32),
        pltpu.VMEM_SHARED((PAD_N, D), jnp.float32),
        pltpu.SemaphoreType.DMA,
        pltpu.SemaphoreType.DMA,
        pltpu.SemaphoreType.DMA,
    ),
)
def _sc_agg(x_hbm, idx_hbm, out_hbm,
            idxA, idxB, buf0, buf1, acc, sem0, sem1, semi):
    c = lax.axis_index("c")
    s = lax.axis_index("s")
    wid = c * NS + s

    pltpu.async_copy(idx_hbm.at[pl.ds(wid * NCH, IB)],
                     idxA.at[pl.ds(0, IB)], semi)
    pltpu.async_copy(idx_hbm.at[pl.ds(HALFROWS + wid * NCH, IB)],
                     idxA.at[pl.ds(IB, IB)], semi)

    def _zrow(i, carry):
        for k in range(D // 16):
            buf0[i, pl.ds(k * 16, 16)] = jnp.zeros((16,), jnp.float32)
        return carry
    lax.fori_loop(0, CB, _zrow, 0)
    nfull = RPT // CB
    for k in range(nfull):
        pltpu.sync_copy(buf0, acc.at[pl.ds(s * RPT + k * CB, CB)])
    rem = RPT - nfull * CB
    if rem:
        pltpu.sync_copy(buf0.at[pl.ds(0, rem)],
                        acc.at[pl.ds(s * RPT + nfull * CB, rem)])
    pltpu.make_async_copy(idx_hbm.at[pl.ds(wid * NCH, IB)],
                          idxA.at[pl.ds(0, IB)], semi).wait()
    pltpu.make_async_copy(idx_hbm.at[pl.ds(HALFROWS + wid * NCH, IB)],
                          idxA.at[pl.ds(IB, IB)], semi).wait()
    plsc.subcore_barrier()

    idx_bufs = (idxA, idxB)
    for b in range(NBLK):
        cur = idx_bufs[b % 2]
        nxt = idx_bufs[(b + 1) % 2]
        row0 = wid * NCH + (b + 1) * IB
        if b + 1 < NBLK:
            pltpu.async_copy(
                idx_hbm.at[pl.ds(row0, IB)], nxt.at[pl.ds(0, IB)], semi)
            pltpu.async_copy(
                idx_hbm.at[pl.ds(HALFROWS + row0, IB)],
                nxt.at[pl.ds(IB, IB)], semi)
        pltpu.async_copy(x_hbm.at[cur.at[0]], buf0, sem0)
        pltpu.async_copy(x_hbm.at[cur.at[1]], buf1, sem1)

        def _body(j2, carry2, cur=cur):
            j = 2 * j2
            pltpu.make_async_copy(x_hbm.at[cur.at[j]], buf0, sem0).wait()
            pltpu.sync_copy(buf0, acc.at[cur.at[IB + j]], add=True)

            @pl.when(j2 < IB // 2 - 1)
            def _():
                pltpu.async_copy(x_hbm.at[cur.at[j + 2]], buf0, sem0)

            pltpu.make_async_copy(x_hbm.at[cur.at[j + 1]], buf1, sem1).wait()
            pltpu.sync_copy(buf1, acc.at[cur.at[IB + j + 1]], add=True)

            @pl.when(j2 < IB // 2 - 1)
            def _():
                pltpu.async_copy(x_hbm.at[cur.at[j + 3]], buf1, sem1)

            return carry2

        lax.fori_loop(0, IB // 2, _body, 0)
        if b + 1 < NBLK:
            pltpu.make_async_copy(
                idx_hbm.at[pl.ds(row0, IB)], nxt.at[pl.ds(0, IB)],
                semi).wait()
            pltpu.make_async_copy(
                idx_hbm.at[pl.ds(HALFROWS + row0, IB)],
                nxt.at[pl.ds(IB, IB)], semi).wait()

    plsc.subcore_barrier()

    pltpu.sync_copy(acc.at[pl.ds(s * RPT, RPT)],
                    out_hbm.at[pl.ds(c * PAD_N + s * RPT, RPT)])


BN = 2000
N_BLK = N // BN


def _dense_body(p0_ref, p1_ref, h_ref, wr_ref, br_ref, wt_ref, o_ref):
    agg = p0_ref[0] + p1_ref[0]
    o = jnp.dot(agg, wr_ref[...], preferred_element_type=jnp.float32)
    o += jnp.dot(h_ref[...], wt_ref[...], preferred_element_type=jnp.float32)
    o_ref[...] = jnp.maximum(o + br_ref[...], 0.0)


def _dense(parts3, h, wr, br, wt):
    return pl.pallas_call(
        _dense_body,
        grid=(N_BLK,),
        in_specs=[
            pl.BlockSpec((1, BN, D), lambda i: (0, i, 0)),
            pl.BlockSpec((1, BN, D), lambda i: (1, i, 0)),
            pl.BlockSpec((BN, D), lambda i: (i, 0)),
            pl.BlockSpec((D, D), lambda i: (0, 0)),
            pl.BlockSpec((1, D), lambda i: (0, 0)),
            pl.BlockSpec((D, D), lambda i: (0, 0)),
        ],
        out_specs=pl.BlockSpec((BN, D), lambda i: (i, 0)),
        out_shape=jax.ShapeDtypeStruct((N, D), jnp.float32),
    )(parts3, parts3, h, wr, br, wt)


def _dense_pool_body(p0_ref, p1_ref, h_ref, wr_ref, br_ref, wt_ref, b_ref,
                     wh_ref, bh_ref, wo_ref, bo_ref, o_ref, g_acc):
    i = pl.program_id(0)

    @pl.when(i == 0)
    def _():
        g_acc[...] = jnp.zeros_like(g_acc)

    agg = p0_ref[0] + p1_ref[0]
    h3 = jnp.dot(agg, wr_ref[...], preferred_element_type=jnp.float32)
    h3 += jnp.dot(h_ref[...], wt_ref[...], preferred_element_type=jnp.float32)
    h3 = jnp.maximum(h3 + br_ref[...], 0.0)

    bb = b_ref[0, 0, :]
    oh = (bb[:, None] == lax.broadcasted_iota(jnp.int32, (BN, G), 1))
    g_acc[...] += lax.dot_general(
        oh.astype(jnp.float32), h3,
        (((0,), (0,)), ((), ())), preferred_element_type=jnp.float32)

    @pl.when(i == N_BLK - 1)
    def _():
        g = jnp.maximum(
            jnp.dot(g_acc[...], wh_ref[...],
                    preferred_element_type=jnp.float32) + bh_ref[...], 0.0)
        o_ref[...] = jnp.dot(
            g, wo_ref[...], preferred_element_type=jnp.float32) + bo_ref[...]


def _dense_pool_mlp(parts3, h, wr, br, wt, batch3, wh, bh, wo, bo):
    return pl.pallas_call(
        _dense_pool_body,
        grid=(N_BLK,),
        in_specs=[
            pl.BlockSpec((1, BN, D), lambda i: (0, i, 0)),
            pl.BlockSpec((1, BN, D), lambda i: (1, i, 0)),
            pl.BlockSpec((BN, D), lambda i: (i, 0)),
            pl.BlockSpec((D, D), lambda i: (0, 0)),
            pl.BlockSpec((1, D), lambda i: (0, 0)),
            pl.BlockSpec((D, D), lambda i: (0, 0)),
            pl.BlockSpec((1, 1, BN), lambda i: (i, 0, 0)),
            pl.BlockSpec((D, D), lambda i: (0, 0)),
            pl.BlockSpec((1, D), lambda i: (0, 0)),
            pl.BlockSpec((D, D), lambda i: (0, 0)),
            pl.BlockSpec((1, D), lambda i: (0, 0)),
        ],
        out_specs=pl.BlockSpec((G, D), lambda i: (0, 0)),
        out_shape=jax.ShapeDtypeStruct((G, D), jnp.float32),
        scratch_shapes=[pltpu.VMEM((G, D), jnp.float32)],
    )(parts3, parts3, h, wr, br, wt, batch3, wh, bh, wo, bo)


@jax.jit
def kernel(x, edge_index, batch, W1_rel, b1_rel, W1_root, W2_rel, b2_rel,
           W2_root, W3_rel, b3_rel, W3_root, Wh, bh, Wo, bo):
    h = x.astype(jnp.float32)
    batch3 = batch.astype(jnp.int32).reshape(N_BLK, 1, BN)
    idx_p = edge_index.astype(jnp.int32).reshape(2 * HALFROWS, CB)

    for wr, br, wt in ((W1_rel, b1_rel, W1_root),
                       (W2_rel, b2_rel, W2_root)):
        parts3 = _sc_agg(h, idx_p).reshape(NC, PAD_N, D)
        h = _dense(parts3, h, wr, br.reshape(1, D), wt)

    parts3 = _sc_agg(h, idx_p).reshape(NC, PAD_N, D)
    return _dense_pool_mlp(parts3, h, W3_rel, b3_rel.reshape(1, D), W3_root,
                           batch3, Wh, bh.reshape(1, D), Wo, bo.reshape(1, D))


# speedup vs baseline: 1.0064x; 1.0064x over previous
import functools

import jax
import jax.numpy as jnp
from jax import lax
from jax.experimental import pallas as pl
from jax.experimental.pallas import tpu as pltpu, tpu_sc as plsc

N = 10000
E = 320000
D = 128
G = 64

PAD_N = 10240
NC = 2
NS = 16
CB = 125
NCH = 80
IB = 16
NBLK = NCH // IB
IDXR = 2 * IB
EPW = NCH * CB
HALFROWS = NC * NS * NCH
RPT = PAD_N // NS

_sc_mesh = plsc.VectorSubcoreMesh(core_axis_name="c", subcore_axis_name="s")


@functools.partial(
    pl.kernel,
    out_type=jax.ShapeDtypeStruct((NC * PAD_N, D), jnp.float32),
    mesh=_sc_mesh,
    compiler_params=pltpu.CompilerParams(use_tc_tiling_on_sc=False),
    scratch_types=(
        pltpu.VMEM((IDXR, CB), jnp.int32),
        pltpu.VMEM((IDXR, CB), jnp.int32),
        pltpu.VMEM((CB, D), jnp.float32),
        pltpu.VMEM((CB, D), jnp.float32),
        pltpu.VMEM_SHARED((PAD_N, D), jnp.float32),
        pltpu.SemaphoreType.DMA,
        pltpu.SemaphoreType.DMA,
        pltpu.SemaphoreType.DMA,
    ),
)
def _sc_agg(x_hbm, idx_hbm, out_hbm,
            idxA, idxB, buf0, buf1, acc, sem0, sem1, semi):
    c = lax.axis_index("c")
    s = lax.axis_index("s")
    wid = c * NS + s

    pltpu.async_copy(idx_hbm.at[pl.ds(wid * NCH, IB)],
                     idxA.at[pl.ds(0, IB)], semi)
    pltpu.async_copy(idx_hbm.at[pl.ds(HALFROWS + wid * NCH, IB)],
                     idxA.at[pl.ds(IB, IB)], semi)

    def _zrow(i, carry):
        for k in range(D // 16):
            buf0[i, pl.ds(k * 16, 16)] = jnp.zeros((16,), jnp.float32)
        return carry
    lax.fori_loop(0, CB, _zrow, 0)
    nfull = RPT // CB
    for k in range(nfull):
        pltpu.sync_copy(buf0, acc.at[pl.ds(s * RPT + k * CB, CB)])
    rem = RPT - nfull * CB
    if rem:
        pltpu.sync_copy(buf0.at[pl.ds(0, rem)],
                        acc.at[pl.ds(s * RPT + nfull * CB, rem)])
    pltpu.make_async_copy(idx_hbm.at[pl.ds(wid * NCH, IB)],
                          idxA.at[pl.ds(0, IB)], semi).wait()
    pltpu.make_async_copy(idx_hbm.at[pl.ds(HALFROWS + wid * NCH, IB)],
                          idxA.at[pl.ds(IB, IB)], semi).wait()
    plsc.subcore_barrier()

    idx_bufs = (idxA, idxB)
    for b in range(NBLK):
        cur = idx_bufs[b % 2]
        nxt = idx_bufs[(b + 1) % 2]
        row0 = wid * NCH + (b + 1) * IB
        if b + 1 < NBLK:
            pltpu.async_copy(
                idx_hbm.at[pl.ds(row0, IB)], nxt.at[pl.ds(0, IB)], semi)
            pltpu.async_copy(
                idx_hbm.at[pl.ds(HALFROWS + row0, IB)],
                nxt.at[pl.ds(IB, IB)], semi)
        pltpu.async_copy(x_hbm.at[cur.at[0]], buf0, sem0)
        pltpu.async_copy(x_hbm.at[cur.at[1]], buf1, sem1)

        def _body(j2, carry2, cur=cur):
            j = 2 * j2
            pltpu.make_async_copy(x_hbm.at[cur.at[j]], buf0, sem0).wait()
            pltpu.sync_copy(buf0, acc.at[cur.at[IB + j]], add=True)

            @pl.when(j2 < IB // 2 - 1)
            def _():
                pltpu.async_copy(x_hbm.at[cur.at[j + 2]], buf0, sem0)

            pltpu.make_async_copy(x_hbm.at[cur.at[j + 1]], buf1, sem1).wait()
            pltpu.sync_copy(buf1, acc.at[cur.at[IB + j + 1]], add=True)

            @pl.when(j2 < IB // 2 - 1)
            def _():
                pltpu.async_copy(x_hbm.at[cur.at[j + 3]], buf1, sem1)

            return carry2

        lax.fori_loop(0, IB // 2, _body, 0)
        if b + 1 < NBLK:
            pltpu.make_async_copy(
                idx_hbm.at[pl.ds(row0, IB)], nxt.at[pl.ds(0, IB)],
                semi).wait()
            pltpu.make_async_copy(
                idx_hbm.at[pl.ds(HALFROWS + row0, IB)],
                nxt.at[pl.ds(IB, IB)], semi).wait()

    plsc.subcore_barrier()

    for k in range(RPT // 128):
        r0 = s * RPT + k * 128
        pltpu.sync_copy(acc.at[pl.ds(r0, 128)],
                        out_hbm.at[pl.ds(c * PAD_N + r0, 128)])


BN = 2000
N_BLK = N // BN


def _dense_body(p0_ref, p1_ref, h_ref, wr_ref, br_ref, wt_ref, o_ref):
    agg = p0_ref[0] + p1_ref[0]
    o = jnp.dot(agg, wr_ref[...], preferred_element_type=jnp.float32)
    o += jnp.dot(h_ref[...], wt_ref[...], preferred_element_type=jnp.float32)
    o_ref[...] = jnp.maximum(o + br_ref[...], 0.0)


def _dense(parts3, h, wr, br, wt):
    return pl.pallas_call(
        _dense_body,
        grid=(N_BLK,),
        in_specs=[
            pl.BlockSpec((1, BN, D), lambda i: (0, i, 0)),
            pl.BlockSpec((1, BN, D), lambda i: (1, i, 0)),
            pl.BlockSpec((BN, D), lambda i: (i, 0)),
            pl.BlockSpec((D, D), lambda i: (0, 0)),
            pl.BlockSpec((1, D), lambda i: (0, 0)),
            pl.BlockSpec((D, D), lambda i: (0, 0)),
        ],
        out_specs=pl.BlockSpec((BN, D), lambda i: (i, 0)),
        out_shape=jax.ShapeDtypeStruct((N, D), jnp.float32),
    )(parts3, parts3, h, wr, br, wt)


def _dense_pool_body(p0_ref, p1_ref, h_ref, wr_ref, br_ref, wt_ref, b_ref,
                     wh_ref, bh_ref, wo_ref, bo_ref, o_ref, g_acc):
    i = pl.program_id(0)

    @pl.when(i == 0)
    def _():
        g_acc[...] = jnp.zeros_like(g_acc)

    agg = p0_ref[0] + p1_ref[0]
    h3 = jnp.dot(agg, wr_ref[...], preferred_element_type=jnp.float32)
    h3 += jnp.dot(h_ref[...], wt_ref[...], preferred_element_type=jnp.float32)
    h3 = jnp.maximum(h3 + br_ref[...], 0.0)

    bb = b_ref[0, 0, :]
    oh = (bb[:, None] == lax.broadcasted_iota(jnp.int32, (BN, G), 1))
    g_acc[...] += lax.dot_general(
        oh.astype(jnp.float32), h3,
        (((0,), (0,)), ((), ())), preferred_element_type=jnp.float32)

    @pl.when(i == N_BLK - 1)
    def _():
        g = jnp.maximum(
            jnp.dot(g_acc[...], wh_ref[...],
                    preferred_element_type=jnp.float32) + bh_ref[...], 0.0)
        o_ref[...] = jnp.dot(
            g, wo_ref[...], preferred_element_type=jnp.float32) + bo_ref[...]


def _dense_pool_mlp(parts3, h, wr, br, wt, batch3, wh, bh, wo, bo):
    return pl.pallas_call(
        _dense_pool_body,
        grid=(N_BLK,),
        in_specs=[
            pl.BlockSpec((1, BN, D), lambda i: (0, i, 0)),
            pl.BlockSpec((1, BN, D), lambda i: (1, i, 0)),
            pl.BlockSpec((BN, D), lambda i: (i, 0)),
            pl.BlockSpec((D, D), lambda i: (0, 0)),
            pl.BlockSpec((1, D), lambda i: (0, 0)),
            pl.BlockSpec((D, D), lambda i: (0, 0)),
            pl.BlockSpec((1, 1, BN), lambda i: (i, 0, 0)),
            pl.BlockSpec((D, D), lambda i: (0, 0)),
            pl.BlockSpec((1, D), lambda i: (0, 0)),
            pl.BlockSpec((D, D), lambda i: (0, 0)),
            pl.BlockSpec((1, D), lambda i: (0, 0)),
        ],
        out_specs=pl.BlockSpec((G, D), lambda i: (0, 0)),
        out_shape=jax.ShapeDtypeStruct((G, D), jnp.float32),
        scratch_shapes=[pltpu.VMEM((G, D), jnp.float32)],
    )(parts3, parts3, h, wr, br, wt, batch3, wh, bh, wo, bo)


@jax.jit
def kernel(x, edge_index, batch, W1_rel, b1_rel, W1_root, W2_rel, b2_rel,
           W2_root, W3_rel, b3_rel, W3_root, Wh, bh, Wo, bo):
    h = x.astype(jnp.float32)
    batch3 = batch.astype(jnp.int32).reshape(N_BLK, 1, BN)
    idx_p = edge_index.astype(jnp.int32).reshape(2 * HALFROWS, CB)

    for wr, br, wt in ((W1_rel, b1_rel, W1_root),
                       (W2_rel, b2_rel, W2_root)):
        parts3 = _sc_agg(h, idx_p).reshape(NC, PAD_N, D)
        h = _dense(parts3, h, wr, br.reshape(1, D), wt)

    parts3 = _sc_agg(h, idx_p).reshape(NC, PAD_N, D)
    return _dense_pool_mlp(parts3, h, W3_rel, b3_rel.reshape(1, D), W3_root,
                           batch3, Wh, bh.reshape(1, D), Wo, bo.reshape(1, D))
